# Initial kernel scaffold; baseline (speedup 1.0000x reference)
import jax
import jax.numpy as jnp
from jax.experimental import pallas as pl


def kernel(x, rbf, i, num_nodes, W_rbfs, W_up, W_layers, b_layers, W_out):
    raise NotImplementedError("write your pallas kernel here")


# speedup vs baseline: 2.6536x; 2.6536x over previous
import functools

import jax
import jax.numpy as jnp
from jax import lax
from jax.experimental import pallas as pl
from jax.experimental.pallas import tpu as pltpu, tpu_sc as plsc

NUM_NODES = 10000
NUM_EDGES = 320000
HIDDEN = 128

NC = 2
NS = 16
EDGES_PER_CORE = NUM_EDGES // NC
EDGES_PER_SUB = EDGES_PER_CORE // NS
CHUNK = 200
NUM_CHUNKS = EDGES_PER_SUB // CHUNK
ROWS_MAIN = 640
ROWS_TAIL = NUM_NODES - (NS - 1) * ROWS_MAIN

EDGE_BLOCK = 4000
NODE_BLOCK = 1000


def _edge_body(rbf_ref, x_ref, wt_ref, t_ref):
    s = jnp.dot(rbf_ref[...], wt_ref[...], preferred_element_type=jnp.float32)
    t_ref[...] = s * x_ref[...]


def _edge_stage(rbf, x, wt):
    grid = (NUM_EDGES // EDGE_BLOCK,)
    return pl.pallas_call(
        _edge_body,
        grid=grid,
        in_specs=[
            pl.BlockSpec((EDGE_BLOCK, rbf.shape[1]), lambda i: (i, 0)),
            pl.BlockSpec((EDGE_BLOCK, HIDDEN), lambda i: (i, 0)),
            pl.BlockSpec(wt.shape, lambda i: (0, 0)),
        ],
        out_specs=pl.BlockSpec((EDGE_BLOCK, HIDDEN), lambda i: (i, 0)),
        out_shape=jax.ShapeDtypeStruct((NUM_EDGES, HIDDEN), jnp.float32),
    )(rbf, x, wt)


def _scatter_body(t_hbm, i_hbm, z_hbm, out_hbm, idx_v, rows_v, acc_sh):
    c = lax.axis_index("c")
    s = lax.axis_index("s")

    @pl.when(s < NS - 1)
    def _():
        pltpu.sync_copy(
            z_hbm.at[pl.ds(s * ROWS_MAIN, ROWS_MAIN)],
            acc_sh.at[pl.ds(s * ROWS_MAIN, ROWS_MAIN)],
        )

    @pl.when(s == NS - 1)
    def _():
        pltpu.sync_copy(
            z_hbm.at[pl.ds((NS - 1) * ROWS_MAIN, ROWS_TAIL)],
            acc_sh.at[pl.ds((NS - 1) * ROWS_MAIN, ROWS_TAIL)],
        )

    plsc.subcore_barrier()

    base0 = c * EDGES_PER_CORE + s * EDGES_PER_SUB

    def body(k, _):
        base = base0 + k * CHUNK
        pltpu.sync_copy(i_hbm.at[pl.ds(base, CHUNK)], idx_v)
        pltpu.sync_copy(t_hbm.at[pl.ds(base, CHUNK)], rows_v)
        pltpu.sync_copy(rows_v, acc_sh.at[idx_v], add=True)
        return _

    lax.fori_loop(0, NUM_CHUNKS, body, None)

    plsc.subcore_barrier()

    @pl.when(s < NS - 1)
    def _():
        pltpu.sync_copy(
            acc_sh.at[pl.ds(s * ROWS_MAIN, ROWS_MAIN)],
            out_hbm.at[c, pl.ds(s * ROWS_MAIN, ROWS_MAIN)],
        )

    @pl.when(s == NS - 1)
    def _():
        pltpu.sync_copy(
            acc_sh.at[pl.ds((NS - 1) * ROWS_MAIN, ROWS_TAIL)],
            out_hbm.at[c, pl.ds((NS - 1) * ROWS_MAIN, ROWS_TAIL)],
        )


_scatter_stage = pl.kernel(
    _scatter_body,
    out_type=jax.ShapeDtypeStruct((NC, NUM_NODES, HIDDEN), jnp.float32),
    mesh=plsc.VectorSubcoreMesh(core_axis_name="c", subcore_axis_name="s"),
    scratch_types=[
        pltpu.VMEM((CHUNK,), jnp.int32),
        pltpu.VMEM((CHUNK, HIDDEN), jnp.float32),
        pltpu.VMEM_SHARED((NUM_NODES, HIDDEN), jnp.float32),
    ],
)


def _mlp_body(parts_ref, wup_ref, wl_ref, bl_ref, wout_ref, out_ref):
    xt = parts_ref[0] + parts_ref[1]
    h = lax.dot_general(
        xt, wup_ref[...], (((1,), (1,)), ((), ())),
        preferred_element_type=jnp.float32,
    )
    for l in range(wl_ref.shape[0]):
        z = lax.dot_general(
            h, wl_ref[l], (((1,), (1,)), ((), ())),
            preferred_element_type=jnp.float32,
        ) + bl_ref[l][None, :]
        h = z * jax.nn.sigmoid(z)
    out_ref[...] = lax.dot_general(
        h, wout_ref[...], (((1,), (1,)), ((), ())),
        preferred_element_type=jnp.float32,
    )


def _mlp_stage(parts, w_up, w_layers, b_layers, w_out):
    grid = (NUM_NODES // NODE_BLOCK,)
    return pl.pallas_call(
        _mlp_body,
        grid=grid,
        in_specs=[
            pl.BlockSpec((NC, NODE_BLOCK, HIDDEN), lambda j: (0, j, 0)),
            pl.BlockSpec(w_up.shape, lambda j: (0, 0)),
            pl.BlockSpec(w_layers.shape, lambda j: (0, 0, 0)),
            pl.BlockSpec(b_layers.shape, lambda j: (0, 0)),
            pl.BlockSpec(w_out.shape, lambda j: (0, 0)),
        ],
        out_specs=pl.BlockSpec((NODE_BLOCK, w_out.shape[0]), lambda j: (j, 0)),
        out_shape=jax.ShapeDtypeStruct((NUM_NODES, w_out.shape[0]), jnp.float32),
    )(parts, w_up, w_layers, b_layers, w_out)


def kernel(x, rbf, i, num_nodes, W_rbfs, W_up, W_layers, b_layers, W_out):
    wt = jnp.transpose(W_rbfs[-1])
    t = _edge_stage(rbf, x, wt)
    zeros = jnp.zeros((NUM_NODES, HIDDEN), jnp.float32)
    parts = _scatter_stage(t, i, zeros)
    return _mlp_stage(parts, W_up, W_layers, b_layers, W_out)
